# Initial kernel scaffold; baseline (speedup 1.0000x reference)
import functools

import jax
import jax.numpy as jnp
from jax import lax
from jax.experimental import pallas as pl
from jax.experimental.pallas import tpu as pltpu
from jax.experimental.pallas import tpu_sc as plsc

N = 10000
E = 320000
D = 128
NC, NS = 2, 16
NW = NC * NS
CHUNK = 128
NCHUNK = 80
EPW = NCHUNK * CHUNK
E_PAD = NW * EPW
N_PAD = 10240
ROWS_PER_TILE = N_PAD // NS

_MESH = plsc.VectorSubcoreMesh(core_axis_name="c", subcore_axis_name="s")


def _zero16():
    return jnp.zeros((16,), jnp.float32)


def _deg_body(row_hbm, deg_out, idx_v, ones_v, zbuf_v, deg_sp):
    cid = lax.axis_index("c")
    sid = lax.axis_index("s")
    wid = cid * NS + sid

    def fill_z(i, _):
        zbuf_v[pl.ds(i * 16, 16)] = _zero16()
        return 0

    lax.fori_loop(0, ROWS_PER_TILE // 16, fill_z, 0)

    def fill_one(i, _):
        ones_v[pl.ds(i * 16, 16)] = jnp.ones((16,), jnp.float32)
        return 0

    lax.fori_loop(0, CHUNK // 16, fill_one, 0)

    base = sid * ROWS_PER_TILE
    pltpu.sync_copy(zbuf_v, deg_sp.at[pl.ds(base, ROWS_PER_TILE)])
    pltpu.sync_copy(row_hbm.at[wid], idx_v)
    plsc.subcore_barrier()

    def scat(j, _):
        pltpu.sync_copy(ones_v, deg_sp.at[idx_v.at[j]], add=True)
        return 0

    lax.fori_loop(0, NCHUNK, scat, 0)
    plsc.subcore_barrier()

    pltpu.sync_copy(deg_sp.at[pl.ds(base, ROWS_PER_TILE)], zbuf_v)
    pltpu.sync_copy(zbuf_v, deg_out.at[cid, pl.ds(base, ROWS_PER_TILE)])


_deg_kernel = functools.partial(
    pl.kernel,
    out_type=jax.ShapeDtypeStruct((NC, N_PAD), jnp.float32),
    mesh=_MESH,
    scratch_types=[
        pltpu.VMEM((NCHUNK, CHUNK), jnp.int32),
        pltpu.VMEM((CHUNK,), jnp.float32),
        pltpu.VMEM((ROWS_PER_TILE,), jnp.float32),
        pltpu.VMEM_SHARED((N_PAD,), jnp.float32),
    ],
)(_deg_body)


def _edge_body(y_hbm, row_hbm, col_hbm, acc_out, idx_r, idx_c, buf, acc_sp, sem):
    cid = lax.axis_index("c")
    sid = lax.axis_index("s")
    wid = cid * NS + sid

    def fill_z(i, _):
        r = i // (D // 16)
        c = i % (D // 16)
        buf[r, pl.ds(c * 16, 16)] = _zero16()
        return 0

    lax.fori_loop(0, CHUNK * (D // 16), fill_z, 0)

    base = sid * ROWS_PER_TILE
    for k in range(ROWS_PER_TILE // CHUNK):
        pltpu.sync_copy(buf, acc_sp.at[pl.ds(base + k * CHUNK, CHUNK)])

    pltpu.sync_copy(row_hbm.at[wid], idx_r)
    pltpu.sync_copy(col_hbm.at[wid], idx_c)
    plsc.subcore_barrier()

    def step(j, _):
        pltpu.async_copy(y_hbm.at[idx_r.at[j]], buf, sem).wait()
        pltpu.sync_copy(buf, acc_sp.at[idx_c.at[j]], add=True)
        return 0

    lax.fori_loop(0, NCHUNK, step, 0)
    plsc.subcore_barrier()

    for k in range(ROWS_PER_TILE // CHUNK):
        pltpu.sync_copy(acc_sp.at[pl.ds(base + k * CHUNK, CHUNK)], buf)
        pltpu.sync_copy(buf, acc_out.at[cid, pl.ds(base + k * CHUNK, CHUNK)])


_edge_kernel = functools.partial(
    pl.kernel,
    out_type=jax.ShapeDtypeStruct((NC, N_PAD, D), jnp.float32),
    mesh=_MESH,
    scratch_types=[
        pltpu.VMEM((NCHUNK, CHUNK), jnp.int32),
        pltpu.VMEM((NCHUNK, CHUNK), jnp.int32),
        pltpu.VMEM((CHUNK, D), jnp.float32),
        pltpu.VMEM_SHARED((N_PAD, D), jnp.float32),
        pltpu.SemaphoreType.DMA,
    ],
)(_edge_body)


def _transform_body(x_ref, w_ref, deg2_ref, y_ref, dis_ref):
    xt = jnp.dot(x_ref[...], w_ref[...], preferred_element_type=jnp.float32)
    deg = deg2_ref[0, :] + deg2_ref[1, :]
    dis = jnp.where(deg > 0.0, lax.rsqrt(deg), 0.0)
    dis_ref[...] = dis
    y_ref[0:N, :] = xt * dis[0:N, None]
    y_ref[N:N_PAD, :] = jnp.zeros((N_PAD - N, D), jnp.float32)


def _finalize_body(acc2_ref, dis_ref, b_ref, out_ref):
    acc = acc2_ref[0, 0:N, :] + acc2_ref[1, 0:N, :]
    out_ref[...] = acc * dis_ref[0:N, None] + b_ref[...][None, :]


def kernel(x, edge_index, W, b):
    row = edge_index[0]
    col = edge_index[1]
    pad = jnp.full((E_PAD - E,), N, jnp.int32)
    row3 = jnp.concatenate([row, pad]).reshape(NW, NCHUNK, CHUNK)
    col3 = jnp.concatenate([col, pad]).reshape(NW, NCHUNK, CHUNK)

    deg2 = _deg_kernel(row3)

    y, dis = pl.pallas_call(
        _transform_body,
        out_shape=(
            jax.ShapeDtypeStruct((N_PAD, D), jnp.float32),
            jax.ShapeDtypeStruct((N_PAD,), jnp.float32),
        ),
    )(x, W, deg2)

    acc2 = _edge_kernel(y, row3, col3)

    out = pl.pallas_call(
        _finalize_body,
        out_shape=jax.ShapeDtypeStruct((N, D), jnp.float32),
    )(acc2, dis, b)
    return out


# speedup vs baseline: 12.7575x; 12.7575x over previous
import functools

import jax
import jax.numpy as jnp
from jax import lax
from jax.experimental import pallas as pl
from jax.experimental.pallas import tpu as pltpu
from jax.experimental.pallas import tpu_sc as plsc

N = 10000
E = 320000
D = 128
NC, NS = 2, 16
NW = NC * NS
CHUNK = 128
NCHUNK = 80
EPW = NCHUNK * CHUNK
E_PAD = NW * EPW
N_PAD = 10240
ROWS_PER_TILE = N_PAD // NS

_MESH = plsc.VectorSubcoreMesh(core_axis_name="c", subcore_axis_name="s")


def _zero16():
    return jnp.zeros((16,), jnp.float32)


def _deg_body(row_hbm, deg_out, idx_v, ones_v, zbuf_v, deg_sp):
    cid = lax.axis_index("c")
    sid = lax.axis_index("s")
    wid = cid * NS + sid

    def fill_z(i, _):
        zbuf_v[pl.ds(i * 16, 16)] = _zero16()
        return 0

    lax.fori_loop(0, ROWS_PER_TILE // 16, fill_z, 0)

    def fill_one(i, _):
        ones_v[pl.ds(i * 16, 16)] = jnp.ones((16,), jnp.float32)
        return 0

    lax.fori_loop(0, CHUNK // 16, fill_one, 0)

    base = sid * ROWS_PER_TILE
    pltpu.sync_copy(zbuf_v, deg_sp.at[pl.ds(base, ROWS_PER_TILE)])
    pltpu.sync_copy(row_hbm.at[wid], idx_v)
    plsc.subcore_barrier()

    def scat(j, _):
        pltpu.sync_copy(ones_v, deg_sp.at[idx_v.at[j]], add=True)
        return 0

    lax.fori_loop(0, NCHUNK, scat, 0)
    plsc.subcore_barrier()

    pltpu.sync_copy(deg_sp.at[pl.ds(base, ROWS_PER_TILE)], zbuf_v)
    pltpu.sync_copy(zbuf_v, deg_out.at[cid, pl.ds(base, ROWS_PER_TILE)])


_deg_kernel = functools.partial(
    pl.kernel,
    out_type=jax.ShapeDtypeStruct((NC, N_PAD), jnp.float32),
    mesh=_MESH,
    scratch_types=[
        pltpu.VMEM((NCHUNK, CHUNK), jnp.int32),
        pltpu.VMEM((CHUNK,), jnp.float32),
        pltpu.VMEM((ROWS_PER_TILE,), jnp.float32),
        pltpu.VMEM_SHARED((N_PAD,), jnp.float32),
    ],
)(_deg_body)


def _edge_body(y_hbm, row_hbm, col_hbm, acc_out, idx_r, idx_c, buf, acc_sp, sem):
    cid = lax.axis_index("c")
    sid = lax.axis_index("s")
    wid = cid * NS + sid

    def fill_z(i, _):
        r = i // (D // 16)
        c = i % (D // 16)
        buf[r, pl.ds(c * 16, 16)] = _zero16()
        return 0

    lax.fori_loop(0, CHUNK * (D // 16), fill_z, 0)

    base = sid * ROWS_PER_TILE
    for k in range(ROWS_PER_TILE // CHUNK):
        pltpu.sync_copy(buf, acc_sp.at[pl.ds(base + k * CHUNK, CHUNK)])

    pltpu.sync_copy(row_hbm.at[wid], idx_r)
    pltpu.sync_copy(col_hbm.at[wid], idx_c)
    plsc.subcore_barrier()

    def step(j, _):
        pltpu.async_copy(y_hbm.at[idx_r.at[j]], buf, sem).wait()
        pltpu.sync_copy(buf, acc_sp.at[idx_c.at[j]], add=True)
        return 0

    lax.fori_loop(0, NCHUNK, step, 0)
    plsc.subcore_barrier()

    for k in range(ROWS_PER_TILE // CHUNK):
        pltpu.sync_copy(acc_sp.at[pl.ds(base + k * CHUNK, CHUNK)], buf)
        pltpu.sync_copy(buf, acc_out.at[cid, pl.ds(base + k * CHUNK, CHUNK)])


_edge_kernel = functools.partial(
    pl.kernel,
    out_type=jax.ShapeDtypeStruct((NC, N_PAD, D), jnp.float32),
    mesh=_MESH,
    scratch_types=[
        pltpu.VMEM((NCHUNK, CHUNK), jnp.int32),
        pltpu.VMEM((NCHUNK, CHUNK), jnp.int32),
        pltpu.VMEM((CHUNK, D), jnp.float32),
        pltpu.VMEM_SHARED((N_PAD, D), jnp.float32),
        pltpu.SemaphoreType.DMA,
    ],
)(_edge_body)


def _transform_body(x_ref, w_ref, deg2_ref, y_ref, dis_ref):
    xt = jnp.dot(x_ref[...], w_ref[...], preferred_element_type=jnp.float32)
    deg = deg2_ref[0, :] + deg2_ref[1, :]
    dis = jnp.where(deg > 0.0, lax.rsqrt(deg), 0.0)
    dis_ref[...] = dis
    y_ref[0:N, :] = xt * dis[0:N, None]
    y_ref[N:N_PAD, :] = jnp.zeros((N_PAD - N, D), jnp.float32)


def _finalize_body(acc2_ref, dis_ref, b_ref, out_ref):
    acc = acc2_ref[0, 0:N, :] + acc2_ref[1, 0:N, :]
    out_ref[...] = acc * dis_ref[0:N][:, None] + b_ref[...][None, :]


def kernel(x, edge_index, W, b):
    row = edge_index[0]
    col = edge_index[1]
    pad = jnp.full((E_PAD - E,), N, jnp.int32)
    row3 = jnp.concatenate([row, pad]).reshape(NW, NCHUNK, CHUNK)
    col3 = jnp.concatenate([col, pad]).reshape(NW, NCHUNK, CHUNK)

    deg2 = _deg_kernel(row3)

    y, dis = pl.pallas_call(
        _transform_body,
        out_shape=(
            jax.ShapeDtypeStruct((N_PAD, D), jnp.float32),
            jax.ShapeDtypeStruct((N_PAD,), jnp.float32),
        ),
    )(x, W, deg2)

    acc2 = _edge_kernel(y, row3, col3)

    out = pl.pallas_call(
        _finalize_body,
        out_shape=jax.ShapeDtypeStruct((N, D), jnp.float32),
    )(acc2, dis, b)
    return out


# speedup vs baseline: 13.7180x; 1.0753x over previous
import functools

import jax
import jax.numpy as jnp
from jax import lax
from jax.experimental import pallas as pl
from jax.experimental.pallas import tpu as pltpu
from jax.experimental.pallas import tpu_sc as plsc

N = 10000
E = 320000
D = 128
NC, NS = 2, 16
NW = NC * NS
CHUNK = 128
NCHUNK = 80
SUP = 16
EPW = NCHUNK * CHUNK
E_PAD = NW * EPW
N_PAD = 10240
ROWS_PER_TILE = N_PAD // NS

_MESH = plsc.VectorSubcoreMesh(core_axis_name="c", subcore_axis_name="s")


def _zero16():
    return jnp.zeros((16,), jnp.float32)


def _deg_body(row_hbm, deg_out, idx_v, ones_v, zbuf_v, deg_sp):
    cid = lax.axis_index("c")
    sid = lax.axis_index("s")
    wid = cid * NS + sid

    def fill_z(i, _):
        zbuf_v[pl.ds(i * 16, 16)] = _zero16()
        return 0

    lax.fori_loop(0, ROWS_PER_TILE // 16, fill_z, 0)

    def fill_one(i, _):
        ones_v[pl.ds(i * 16, 16)] = jnp.ones((16,), jnp.float32)
        return 0

    lax.fori_loop(0, CHUNK // 16, fill_one, 0)

    base = sid * ROWS_PER_TILE
    pltpu.sync_copy(zbuf_v, deg_sp.at[pl.ds(base, ROWS_PER_TILE)])
    pltpu.sync_copy(row_hbm.at[wid], idx_v)
    plsc.subcore_barrier()

    def scat(j, _):
        pltpu.sync_copy(ones_v, deg_sp.at[idx_v.at[j]], add=True)
        return 0

    lax.fori_loop(0, NCHUNK, scat, 0)
    plsc.subcore_barrier()

    pltpu.sync_copy(deg_sp.at[pl.ds(base, ROWS_PER_TILE)], zbuf_v)
    pltpu.sync_copy(zbuf_v, deg_out.at[cid, pl.ds(base, ROWS_PER_TILE)])


_deg_kernel = functools.partial(
    pl.kernel,
    out_type=jax.ShapeDtypeStruct((NC, N_PAD), jnp.float32),
    mesh=_MESH,
    scratch_types=[
        pltpu.VMEM((NCHUNK, CHUNK), jnp.int32),
        pltpu.VMEM((CHUNK,), jnp.float32),
        pltpu.VMEM((ROWS_PER_TILE,), jnp.float32),
        pltpu.VMEM_SHARED((N_PAD,), jnp.float32),
    ],
)(_deg_body)


def _edge_body(y_hbm, row_hbm, col_hbm, acc_out, idx_r, idx_c, buf0, buf1,
               acc_sp, sem0, sem1):
    cid = lax.axis_index("c")
    sid = lax.axis_index("s")
    wid = cid * NS + sid

    def fill_z(i, _):
        r = i // (D // 16)
        c = i % (D // 16)
        buf0[r, pl.ds(c * 16, 16)] = _zero16()
        return 0

    lax.fori_loop(0, CHUNK * (D // 16), fill_z, 0)

    base = sid * ROWS_PER_TILE
    for k in range(ROWS_PER_TILE // CHUNK):
        pltpu.sync_copy(buf0, acc_sp.at[pl.ds(base + k * CHUNK, CHUNK)])

    plsc.subcore_barrier()

    def sup(s, _):
        pltpu.sync_copy(row_hbm.at[wid, pl.ds(s * SUP, SUP)], idx_r)
        pltpu.sync_copy(col_hbm.at[wid, pl.ds(s * SUP, SUP)], idx_c)
        pltpu.async_copy(y_hbm.at[idx_r.at[0]], buf0, sem0)

        def step(t, _):
            j0 = 2 * t
            pltpu.make_async_copy(y_hbm.at[idx_r.at[j0]], buf0, sem0).wait()
            pltpu.async_copy(y_hbm.at[idx_r.at[j0 + 1]], buf1, sem1)
            pltpu.sync_copy(buf0, acc_sp.at[idx_c.at[j0]], add=True)
            pltpu.make_async_copy(y_hbm.at[idx_r.at[j0 + 1]], buf1, sem1).wait()

            @pl.when(j0 + 2 < SUP)
            def _():
                pltpu.async_copy(y_hbm.at[idx_r.at[j0 + 2]], buf0, sem0)

            pltpu.sync_copy(buf1, acc_sp.at[idx_c.at[j0 + 1]], add=True)
            return 0

        lax.fori_loop(0, SUP // 2, step, 0)
        return 0

    lax.fori_loop(0, NCHUNK // SUP, sup, 0)
    plsc.subcore_barrier()

    for k in range(ROWS_PER_TILE // CHUNK):
        pltpu.sync_copy(acc_sp.at[pl.ds(base + k * CHUNK, CHUNK)], buf0)
        pltpu.sync_copy(buf0, acc_out.at[cid, pl.ds(base + k * CHUNK, CHUNK)])


_edge_kernel = functools.partial(
    pl.kernel,
    out_type=jax.ShapeDtypeStruct((NC, N_PAD, D), jnp.float32),
    mesh=_MESH,
    scratch_types=[
        pltpu.VMEM((SUP, CHUNK), jnp.int32),
        pltpu.VMEM((SUP, CHUNK), jnp.int32),
        pltpu.VMEM((CHUNK, D), jnp.float32),
        pltpu.VMEM((CHUNK, D), jnp.float32),
        pltpu.VMEM_SHARED((N_PAD, D), jnp.float32),
        pltpu.SemaphoreType.DMA,
        pltpu.SemaphoreType.DMA,
    ],
)(_edge_body)


def _transform_body(x_ref, w_ref, deg2_ref, y_ref, dis_ref):
    xt = jnp.dot(x_ref[...], w_ref[...], preferred_element_type=jnp.float32)
    deg = deg2_ref[0, :] + deg2_ref[1, :]
    dis = jnp.where(deg > 0.0, lax.rsqrt(deg), 0.0)
    dis_ref[...] = dis
    y_ref[0:N, :] = xt * dis[0:N, None]
    y_ref[N:N_PAD, :] = jnp.zeros((N_PAD - N, D), jnp.float32)


def _finalize_body(acc2_ref, dis_ref, b_ref, out_ref):
    acc = acc2_ref[0, 0:N, :] + acc2_ref[1, 0:N, :]
    out_ref[...] = acc * dis_ref[0:N][:, None] + b_ref[...][None, :]


def kernel(x, edge_index, W, b):
    row = edge_index[0]
    col = edge_index[1]
    pad = jnp.full((E_PAD - E,), N, jnp.int32)
    row3 = jnp.concatenate([row, pad]).reshape(NW, NCHUNK, CHUNK)
    col3 = jnp.concatenate([col, pad]).reshape(NW, NCHUNK, CHUNK)

    deg2 = _deg_kernel(row3)

    y, dis = pl.pallas_call(
        _transform_body,
        out_shape=(
            jax.ShapeDtypeStruct((N_PAD, D), jnp.float32),
            jax.ShapeDtypeStruct((N_PAD,), jnp.float32),
        ),
    )(x, W, deg2)

    acc2 = _edge_kernel(y, row3, col3)

    out = pl.pallas_call(
        _finalize_body,
        out_shape=jax.ShapeDtypeStruct((N, D), jnp.float32),
    )(acc2, dis, b)
    return out


# speedup vs baseline: 34.8367x; 2.5395x over previous
import functools

import jax
import jax.numpy as jnp
from jax import lax
from jax.experimental import pallas as pl
from jax.experimental.pallas import tpu as pltpu
from jax.experimental.pallas import tpu_sc as plsc

N = 10000
E = 320000
D = 128
NC, NS = 2, 16
NW = NC * NS
CHUNK = 128
NCHUNK = 80
SUP = 16
EPW = NCHUNK * CHUNK
E_PAD = NW * EPW
N_PAD = 10240
ROWS_PER_TILE = N_PAD // NS

_MESH = plsc.VectorSubcoreMesh(core_axis_name="c", subcore_axis_name="s")


def _zero16():
    return jnp.zeros((16,), jnp.float32)


def _deg_body(row_hbm, deg_out, idx_v, ones_v, zbuf_v, deg_sp):
    cid = lax.axis_index("c")
    sid = lax.axis_index("s")
    wid = cid * NS + sid

    def fill_z(i, _):
        zbuf_v[pl.ds(i * 16, 16)] = _zero16()
        return 0

    lax.fori_loop(0, ROWS_PER_TILE // 16, fill_z, 0)

    def fill_one(i, _):
        ones_v[pl.ds(i * 16, 16)] = jnp.ones((16,), jnp.float32)
        return 0

    lax.fori_loop(0, CHUNK // 16, fill_one, 0)

    base = sid * ROWS_PER_TILE
    pltpu.sync_copy(zbuf_v, deg_sp.at[pl.ds(base, ROWS_PER_TILE)])
    pltpu.sync_copy(row_hbm.at[wid], idx_v)
    plsc.subcore_barrier()

    def scat(j, _):
        pltpu.sync_copy(ones_v, deg_sp.at[idx_v.at[j]], add=True)
        return 0

    lax.fori_loop(0, NCHUNK, scat, 0)
    plsc.subcore_barrier()

    pltpu.sync_copy(deg_sp.at[pl.ds(base, ROWS_PER_TILE)], zbuf_v)
    pltpu.sync_copy(zbuf_v, deg_out.at[cid, pl.ds(base, ROWS_PER_TILE)])


_deg_kernel = functools.partial(
    pl.kernel,
    out_type=jax.ShapeDtypeStruct((NC, N_PAD), jnp.float32),
    mesh=_MESH,
    scratch_types=[
        pltpu.VMEM((NCHUNK, CHUNK), jnp.int32),
        pltpu.VMEM((CHUNK,), jnp.float32),
        pltpu.VMEM((ROWS_PER_TILE,), jnp.float32),
        pltpu.VMEM_SHARED((N_PAD,), jnp.float32),
    ],
)(_deg_body)


def _edge_body(y_hbm, row_hbm, col_hbm, acc_out, idx_r, idx_c, buf0, buf1,
               acc_sp, sem0, sem1):
    cid = lax.axis_index("c")
    sid = lax.axis_index("s")
    wid = cid * NS + sid

    def fill_z(i, _):
        r = i // (D // 16)
        c = i % (D // 16)
        buf0[r, pl.ds(c * 16, 16)] = _zero16()
        return 0

    lax.fori_loop(0, CHUNK * (D // 16), fill_z, 0)

    base = sid * ROWS_PER_TILE
    for k in range(ROWS_PER_TILE // CHUNK):
        pltpu.sync_copy(buf0, acc_sp.at[pl.ds(base + k * CHUNK, CHUNK)])

    plsc.subcore_barrier()

    def sup(s, _):
        pltpu.sync_copy(row_hbm.at[wid, pl.ds(s * SUP, SUP)], idx_r)
        pltpu.sync_copy(col_hbm.at[wid, pl.ds(s * SUP, SUP)], idx_c)
        pltpu.async_copy(y_hbm.at[idx_r.at[0]], buf0, sem0)

        def step(t, _):
            j0 = 2 * t
            pltpu.make_async_copy(y_hbm.at[idx_r.at[j0]], buf0, sem0).wait()
            pltpu.async_copy(y_hbm.at[idx_r.at[j0 + 1]], buf1, sem1)
            pltpu.sync_copy(buf0, acc_sp.at[idx_c.at[j0]], add=True)
            pltpu.make_async_copy(y_hbm.at[idx_r.at[j0 + 1]], buf1, sem1).wait()

            @pl.when(j0 + 2 < SUP)
            def _():
                pltpu.async_copy(y_hbm.at[idx_r.at[j0 + 2]], buf0, sem0)

            pltpu.sync_copy(buf1, acc_sp.at[idx_c.at[j0 + 1]], add=True)
            return 0

        lax.fori_loop(0, SUP // 2, step, 0)
        return 0

    lax.fori_loop(0, NCHUNK // SUP, sup, 0)
    plsc.subcore_barrier()

    for k in range(ROWS_PER_TILE // CHUNK):
        pltpu.sync_copy(acc_sp.at[pl.ds(base + k * CHUNK, CHUNK)], buf0)
        pltpu.sync_copy(buf0, acc_out.at[cid, pl.ds(base + k * CHUNK, CHUNK)])


_edge_kernel = functools.partial(
    pl.kernel,
    out_type=jax.ShapeDtypeStruct((NC, N_PAD, D), jnp.float32),
    mesh=_MESH,
    scratch_types=[
        pltpu.VMEM((SUP, CHUNK), jnp.int32),
        pltpu.VMEM((SUP, CHUNK), jnp.int32),
        pltpu.VMEM((CHUNK, D), jnp.float32),
        pltpu.VMEM((CHUNK, D), jnp.float32),
        pltpu.VMEM_SHARED((N_PAD, D), jnp.float32),
        pltpu.SemaphoreType.DMA,
        pltpu.SemaphoreType.DMA,
    ],
)(_edge_body)


def _transform_body(x_ref, w_ref, deg2_ref, y_ref, dis_ref):
    xt = jnp.dot(x_ref[...], w_ref[...], preferred_element_type=jnp.float32)
    deg = deg2_ref[0, :] + deg2_ref[1, :]
    dis = jnp.where(deg > 0.0, lax.rsqrt(deg), 0.0)
    dis_ref[...] = dis
    y_ref[0:N, :] = xt * dis[0:N, None]
    y_ref[N:N_PAD, :] = jnp.zeros((N_PAD - N, D), jnp.float32)


def _finalize_body(acc2_ref, dis_ref, b_ref, out_ref):
    acc = acc2_ref[0, 0:N, :] + acc2_ref[1, 0:N, :]
    out_ref[...] = acc * dis_ref[0:N][:, None] + b_ref[...][None, :]


def kernel(x, edge_index, W, b):
    row = edge_index[0]
    col = edge_index[1]
    pad = N + jnp.arange(E_PAD - E, dtype=jnp.int32) % (N_PAD - N)
    row3 = jnp.concatenate([row, pad]).reshape(NW, NCHUNK, CHUNK)
    col3 = jnp.concatenate([col, pad]).reshape(NW, NCHUNK, CHUNK)

    deg2 = _deg_kernel(row3)

    y, dis = pl.pallas_call(
        _transform_body,
        out_shape=(
            jax.ShapeDtypeStruct((N_PAD, D), jnp.float32),
            jax.ShapeDtypeStruct((N_PAD,), jnp.float32),
        ),
    )(x, W, deg2)

    acc2 = _edge_kernel(y, row3, col3)

    out = pl.pallas_call(
        _finalize_body,
        out_shape=jax.ShapeDtypeStruct((N, D), jnp.float32),
    )(acc2, dis, b)
    return out


# speedup vs baseline: 40.7004x; 1.1683x over previous
import functools

import jax
import jax.numpy as jnp
from jax import lax
from jax.experimental import pallas as pl
from jax.experimental.pallas import tpu as pltpu
from jax.experimental.pallas import tpu_sc as plsc

N = 10000
E = 320000
D = 128
NC, NS = 2, 16
NW = NC * NS
CHUNK = 128
NCHUNK = 80
SUP = 40
EPW = NCHUNK * CHUNK
E_PAD = NW * EPW
N_PAD = 10240
ROWS_PER_TILE = N_PAD // NS

_MESH = plsc.VectorSubcoreMesh(core_axis_name="c", subcore_axis_name="s")


def _zero16():
    return jnp.zeros((16,), jnp.float32)


def _deg_body(row_hbm, deg_out, idx_v, ones_v, zbuf_v, deg_sp):
    cid = lax.axis_index("c")
    sid = lax.axis_index("s")
    wid = cid * NS + sid

    def fill_z(i, _):
        zbuf_v[pl.ds(i * 16, 16)] = _zero16()
        return 0

    lax.fori_loop(0, ROWS_PER_TILE // 16, fill_z, 0)

    def fill_one(i, _):
        ones_v[pl.ds(i * 16, 16)] = jnp.ones((16,), jnp.float32)
        return 0

    lax.fori_loop(0, CHUNK // 16, fill_one, 0)

    base = sid * ROWS_PER_TILE
    pltpu.sync_copy(zbuf_v, deg_sp.at[pl.ds(base, ROWS_PER_TILE)])
    pltpu.sync_copy(row_hbm.at[wid], idx_v)
    plsc.subcore_barrier()

    def scat(j, _):
        pltpu.sync_copy(ones_v, deg_sp.at[idx_v.at[j]], add=True)
        return 0

    lax.fori_loop(0, NCHUNK, scat, 0)
    plsc.subcore_barrier()

    pltpu.sync_copy(deg_sp.at[pl.ds(base, ROWS_PER_TILE)], zbuf_v)
    pltpu.sync_copy(zbuf_v, deg_out.at[cid, pl.ds(base, ROWS_PER_TILE)])


_deg_kernel = functools.partial(
    pl.kernel,
    out_type=jax.ShapeDtypeStruct((NC, N_PAD), jnp.float32),
    mesh=_MESH,
    scratch_types=[
        pltpu.VMEM((NCHUNK, CHUNK), jnp.int32),
        pltpu.VMEM((CHUNK,), jnp.float32),
        pltpu.VMEM((ROWS_PER_TILE,), jnp.float32),
        pltpu.VMEM_SHARED((N_PAD,), jnp.float32),
    ],
)(_deg_body)


def _edge_body(y_hbm, row_hbm, col_hbm, acc_out, idx_r, idx_c, buf0, buf1,
               acc_sp, sem0, sem1):
    cid = lax.axis_index("c")
    sid = lax.axis_index("s")
    wid = cid * NS + sid

    def fill_z(i, _):
        r = i // (D // 16)
        c = i % (D // 16)
        buf0[r, pl.ds(c * 16, 16)] = _zero16()
        return 0

    lax.fori_loop(0, CHUNK * (D // 16), fill_z, 0)

    base = sid * ROWS_PER_TILE
    for k in range(ROWS_PER_TILE // CHUNK):
        pltpu.sync_copy(buf0, acc_sp.at[pl.ds(base + k * CHUNK, CHUNK)])

    plsc.subcore_barrier()

    def sup(s, _):
        pltpu.sync_copy(row_hbm.at[wid, pl.ds(s * SUP, SUP)], idx_r)
        pltpu.sync_copy(col_hbm.at[wid, pl.ds(s * SUP, SUP)], idx_c)
        pltpu.async_copy(y_hbm.at[idx_r.at[0]], buf0, sem0)
        pltpu.async_copy(y_hbm.at[idx_r.at[1]], buf1, sem1)

        def step(t, _):
            j0 = 2 * t
            pltpu.make_async_copy(y_hbm.at[idx_r.at[j0]], buf0, sem0).wait()
            pltpu.sync_copy(buf0, acc_sp.at[idx_c.at[j0]], add=True)

            @pl.when(j0 + 2 < SUP)
            def _():
                pltpu.async_copy(y_hbm.at[idx_r.at[j0 + 2]], buf0, sem0)

            pltpu.make_async_copy(y_hbm.at[idx_r.at[j0 + 1]], buf1, sem1).wait()
            pltpu.sync_copy(buf1, acc_sp.at[idx_c.at[j0 + 1]], add=True)

            @pl.when(j0 + 3 < SUP)
            def _():
                pltpu.async_copy(y_hbm.at[idx_r.at[j0 + 3]], buf1, sem1)

            return 0

        lax.fori_loop(0, SUP // 2, step, 0)
        return 0

    lax.fori_loop(0, NCHUNK // SUP, sup, 0)
    plsc.subcore_barrier()

    for k in range(ROWS_PER_TILE // CHUNK):
        pltpu.sync_copy(acc_sp.at[pl.ds(base + k * CHUNK, CHUNK)], buf0)
        pltpu.sync_copy(buf0, acc_out.at[cid, pl.ds(base + k * CHUNK, CHUNK)])


_edge_kernel = functools.partial(
    pl.kernel,
    out_type=jax.ShapeDtypeStruct((NC, N_PAD, D), jnp.float32),
    mesh=_MESH,
    scratch_types=[
        pltpu.VMEM((SUP, CHUNK), jnp.int32),
        pltpu.VMEM((SUP, CHUNK), jnp.int32),
        pltpu.VMEM((CHUNK, D), jnp.float32),
        pltpu.VMEM((CHUNK, D), jnp.float32),
        pltpu.VMEM_SHARED((N_PAD, D), jnp.float32),
        pltpu.SemaphoreType.DMA,
        pltpu.SemaphoreType.DMA,
    ],
)(_edge_body)


def _transform_body(x_ref, w_ref, deg2_ref, y_ref, dis_ref):
    xt = jnp.dot(x_ref[...], w_ref[...], preferred_element_type=jnp.float32)
    deg = deg2_ref[0, :] + deg2_ref[1, :]
    dis = jnp.where(deg > 0.0, lax.rsqrt(deg), 0.0)
    dis_ref[...] = dis
    y_ref[0:N, :] = xt * dis[0:N, None]
    y_ref[N:N_PAD, :] = jnp.zeros((N_PAD - N, D), jnp.float32)


def _finalize_body(acc2_ref, dis_ref, b_ref, out_ref):
    acc = acc2_ref[0, 0:N, :] + acc2_ref[1, 0:N, :]
    out_ref[...] = acc * dis_ref[0:N][:, None] + b_ref[...][None, :]


def kernel(x, edge_index, W, b):
    row = edge_index[0]
    col = edge_index[1]
    pad = N + jnp.arange(E_PAD - E, dtype=jnp.int32) % (N_PAD - N)
    row3 = jnp.concatenate([row, pad]).reshape(NW, NCHUNK, CHUNK)
    col3 = jnp.concatenate([col, pad]).reshape(NW, NCHUNK, CHUNK)

    deg2 = _deg_kernel(row3)

    y, dis = pl.pallas_call(
        _transform_body,
        out_shape=(
            jax.ShapeDtypeStruct((N_PAD, D), jnp.float32),
            jax.ShapeDtypeStruct((N_PAD,), jnp.float32),
        ),
    )(x, W, deg2)

    acc2 = _edge_kernel(y, row3, col3)

    out = pl.pallas_call(
        _finalize_body,
        out_shape=jax.ShapeDtypeStruct((N, D), jnp.float32),
    )(acc2, dis, b)
    return out


# speedup vs baseline: 43.3532x; 1.0652x over previous
import functools

import jax
import jax.numpy as jnp
from jax import lax
from jax.experimental import pallas as pl
from jax.experimental.pallas import tpu as pltpu
from jax.experimental.pallas import tpu_sc as plsc

N = 10000
E = 320000
D = 128
NC, NS = 2, 16
NW = NC * NS
CHUNK = 128
NCHUNK = 80
SUP = 40
EPW = NCHUNK * CHUNK
E_PAD = NW * EPW
N_PAD = 10240
ROWS_PER_TILE = N_PAD // NS

_MESH = plsc.VectorSubcoreMesh(core_axis_name="c", subcore_axis_name="s")


def _zero16():
    return jnp.zeros((16,), jnp.float32)


def _deg_body(rc_hbm, deg_out, idx_v, ones_v, zbuf_v, deg_sp, sem):
    cid = lax.axis_index("c")
    sid = lax.axis_index("s")
    wid = cid * NS + sid

    def fill_z(i, _):
        zbuf_v[pl.ds(i * 16, 16)] = _zero16()
        return 0

    lax.fori_loop(0, ROWS_PER_TILE // 16, fill_z, 0)

    def fill_one(i, _):
        ones_v[pl.ds(i * 16, 16)] = jnp.ones((16,), jnp.float32)
        return 0

    lax.fori_loop(0, CHUNK // 16, fill_one, 0)

    base = sid * ROWS_PER_TILE
    pltpu.sync_copy(zbuf_v, deg_sp.at[pl.ds(base, ROWS_PER_TILE)])
    pltpu.sync_copy(rc_hbm.at[0, wid], idx_v)
    plsc.subcore_barrier()

    def scat(j, _):
        pltpu.async_copy(ones_v, deg_sp.at[idx_v.at[j]], sem, add=True)
        return 0

    lax.fori_loop(0, NCHUNK, scat, 0)

    def drain(j, _):
        pltpu.make_async_copy(ones_v, deg_sp.at[idx_v.at[j]], sem).wait()
        return 0

    lax.fori_loop(0, NCHUNK, drain, 0)
    plsc.subcore_barrier()

    pltpu.sync_copy(deg_sp.at[pl.ds(base, ROWS_PER_TILE)], zbuf_v)
    pltpu.sync_copy(zbuf_v, deg_out.at[cid, pl.ds(base, ROWS_PER_TILE)])


_deg_kernel = functools.partial(
    pl.kernel,
    out_type=jax.ShapeDtypeStruct((NC, N_PAD), jnp.float32),
    mesh=_MESH,
    scratch_types=[
        pltpu.VMEM((NCHUNK, CHUNK), jnp.int32),
        pltpu.VMEM((CHUNK,), jnp.float32),
        pltpu.VMEM((ROWS_PER_TILE,), jnp.float32),
        pltpu.VMEM_SHARED((N_PAD,), jnp.float32),
        pltpu.SemaphoreType.DMA,
    ],
)(_deg_body)


def _edge_body(y_hbm, rc_hbm, acc_out, idx_r, idx_c, buf0, buf1,
               acc_sp, sem0, sem1):
    cid = lax.axis_index("c")
    sid = lax.axis_index("s")
    wid = cid * NS + sid

    def fill_z(i, _):
        r = i // (D // 16)
        c = i % (D // 16)
        buf0[r, pl.ds(c * 16, 16)] = _zero16()
        return 0

    lax.fori_loop(0, CHUNK * (D // 16), fill_z, 0)

    base = sid * ROWS_PER_TILE
    for k in range(ROWS_PER_TILE // CHUNK):
        pltpu.sync_copy(buf0, acc_sp.at[pl.ds(base + k * CHUNK, CHUNK)])

    plsc.subcore_barrier()

    def sup(s, _):
        pltpu.sync_copy(rc_hbm.at[0, wid, pl.ds(s * SUP, SUP)], idx_r)
        pltpu.sync_copy(rc_hbm.at[1, wid, pl.ds(s * SUP, SUP)], idx_c)
        pltpu.async_copy(y_hbm.at[idx_r.at[0]], buf0, sem0)
        pltpu.async_copy(y_hbm.at[idx_r.at[1]], buf1, sem1)

        def step(t, _):
            j0 = 2 * t
            pltpu.make_async_copy(y_hbm.at[idx_r.at[j0]], buf0, sem0).wait()
            pltpu.sync_copy(buf0, acc_sp.at[idx_c.at[j0]], add=True)

            @pl.when(j0 + 2 < SUP)
            def _():
                pltpu.async_copy(y_hbm.at[idx_r.at[j0 + 2]], buf0, sem0)

            pltpu.make_async_copy(y_hbm.at[idx_r.at[j0 + 1]], buf1, sem1).wait()
            pltpu.sync_copy(buf1, acc_sp.at[idx_c.at[j0 + 1]], add=True)

            @pl.when(j0 + 3 < SUP)
            def _():
                pltpu.async_copy(y_hbm.at[idx_r.at[j0 + 3]], buf1, sem1)

            return 0

        lax.fori_loop(0, SUP // 2, step, 0)
        return 0

    lax.fori_loop(0, NCHUNK // SUP, sup, 0)
    plsc.subcore_barrier()

    for k in range(ROWS_PER_TILE // CHUNK):
        pltpu.sync_copy(acc_sp.at[pl.ds(base + k * CHUNK, CHUNK)], buf0)
        pltpu.sync_copy(buf0, acc_out.at[cid, pl.ds(base + k * CHUNK, CHUNK)])


_edge_kernel = functools.partial(
    pl.kernel,
    out_type=jax.ShapeDtypeStruct((NC, N_PAD, D), jnp.float32),
    mesh=_MESH,
    scratch_types=[
        pltpu.VMEM((SUP, CHUNK), jnp.int32),
        pltpu.VMEM((SUP, CHUNK), jnp.int32),
        pltpu.VMEM((CHUNK, D), jnp.float32),
        pltpu.VMEM((CHUNK, D), jnp.float32),
        pltpu.VMEM_SHARED((N_PAD, D), jnp.float32),
        pltpu.SemaphoreType.DMA,
        pltpu.SemaphoreType.DMA,
    ],
)(_edge_body)


def _matmul_body(x_ref, w_ref, xt_ref):
    xt_ref[...] = jnp.dot(x_ref[...], w_ref[...],
                          preferred_element_type=jnp.float32)


def _scale_body(xt_ref, deg2_ref, y_ref, dis_ref):
    deg = deg2_ref[0, :] + deg2_ref[1, :]
    dis = jnp.where(deg > 0.0, lax.rsqrt(deg), 0.0)
    dis_ref[...] = dis
    y_ref[0:N, :] = xt_ref[...] * dis[0:N, None]
    y_ref[N:N_PAD, :] = jnp.zeros((N_PAD - N, D), jnp.float32)


def _finalize_body(acc2_ref, dis_ref, b_ref, out_ref):
    acc = acc2_ref[0, 0:N, :] + acc2_ref[1, 0:N, :]
    out_ref[...] = acc * dis_ref[0:N][:, None] + b_ref[...][None, :]


def kernel(x, edge_index, W, b):
    pad = N + jnp.arange(E_PAD - E, dtype=jnp.int32) % (N_PAD - N)
    rc4 = jnp.concatenate(
        [edge_index, jnp.broadcast_to(pad, (2, E_PAD - E))], axis=1
    ).reshape(2, NW, NCHUNK, CHUNK)

    deg2 = _deg_kernel(rc4)

    xt = pl.pallas_call(
        _matmul_body,
        out_shape=jax.ShapeDtypeStruct((N, D), jnp.float32),
    )(x, W)

    y, dis = pl.pallas_call(
        _scale_body,
        out_shape=(
            jax.ShapeDtypeStruct((N_PAD, D), jnp.float32),
            jax.ShapeDtypeStruct((N_PAD,), jnp.float32),
        ),
    )(xt, deg2)

    acc2 = _edge_kernel(y, rc4)

    out = pl.pallas_call(
        _finalize_body,
        out_shape=jax.ShapeDtypeStruct((N, D), jnp.float32),
    )(acc2, dis, b)
    return out
